# Initial kernel scaffold; baseline (speedup 1.0000x reference)
import jax
import jax.numpy as jnp
from jax.experimental import pallas as pl


def kernel(x, edge_index, W1, a_src1, a_dst1, b1, W2, a_src2, a_dst2, b2):
    raise NotImplementedError("write your pallas kernel here")


# speedup vs baseline: 1.0005x; 1.0005x over previous
import jax
import jax.numpy as jnp
from jax.experimental import pallas as pl

N_NODES = 10000


def _gat_conv(x, src, dst, W, a_src, a_dst, b, num_nodes):
    h = x @ W
    alpha_s = (h * a_src).sum(axis=-1)
    alpha_d = (h * a_dst).sum(axis=-1)
    e = alpha_s[src] + alpha_d[dst]
    e = jax.nn.leaky_relu(e, negative_slope=0.2)
    m = jax.ops.segment_max(e, dst, num_segments=num_nodes)
    e = jnp.exp(e - m[dst])
    denom = jax.ops.segment_sum(e, dst, num_segments=num_nodes)
    coef = e / denom[dst]
    out = jax.ops.segment_sum(h[src] * coef[:, None], dst, num_segments=num_nodes)
    return out + b


def _logsoftmax_kernel(x_ref, o_ref):
    x = x_ref[...]
    m = jnp.max(x, axis=-1, keepdims=True)
    s = jnp.log(jnp.sum(jnp.exp(x - m), axis=-1, keepdims=True))
    o_ref[...] = x - m - s


def kernel(x, edge_index, W1, a_src1, a_dst1, b1, W2, a_src2, a_dst2, b2):
    src = edge_index[0]
    dst = edge_index[1]
    h = _gat_conv(x, src, dst, W1, a_src1, a_dst1, b1, N_NODES)
    h = jax.nn.relu(h)
    h = _gat_conv(h, src, dst, W2, a_src2, a_dst2, b2, N_NODES)
    return pl.pallas_call(
        _logsoftmax_kernel,
        out_shape=jax.ShapeDtypeStruct(h.shape, h.dtype),
    )(h)


# speedup vs baseline: 29.4234x; 29.4094x over previous
import jax
import jax.numpy as jnp
from jax import lax
from jax.experimental import pallas as pl
from jax.experimental.pallas import tpu as pltpu
from jax.experimental.pallas import tpu_sc as plsc

N_NODES = 10000
N_EDGES = 320000
D_IN = 128
D_HID = 8
N_CLASSES = 10

NPAD = 10240
DUMMY = N_NODES
NS = 16
NW = 32
NCH = 80
CW = 128
EW = NCH * CW
PAD_E = NW * EW
ROWS_PER_TILE = NPAD // NS
DW = 16

_f32 = jnp.float32
_i32 = jnp.int32

_GATHER_DNUMS = lax.GatherDimensionNumbers(
    offset_dims=(), collapsed_slice_dims=(0,), start_index_map=(0,))


def _splat_lane(vec16, j):
    idx = jnp.full((16, 1), j, _i32)
    return lax.gather(vec16, idx, _GATHER_DNUMS, slice_sizes=(1,),
                      mode=lax.GatherScatterMode.PROMISE_IN_BOUNDS)


def _tc1_body(x_ref, w_ref, aws_ref, awd_ref,
              h_ref, as_ref, ad_ref, bs_ref, bd_ref):
    i = pl.program_id(0)
    h = jnp.dot(x_ref[...], w_ref[...], preferred_element_type=_f32)
    h_ref[...] = jnp.concatenate(
        [h, jnp.zeros((h.shape[0], DW - D_HID), _f32)], axis=1)
    a_s = jnp.sum(h * aws_ref[...][None, :], axis=1)
    a_d = jnp.sum(h * awd_ref[...][None, :], axis=1)
    as_ref[...] = a_s
    ad_ref[...] = a_d
    ms = jnp.full((8, 128), jnp.max(a_s), _f32)
    md = jnp.full((8, 128), jnp.max(a_d), _f32)

    @pl.when(i == 0)
    def _():
        bs_ref[...] = ms
        bd_ref[...] = md

    @pl.when(i > 0)
    def _():
        bs_ref[...] = jnp.maximum(bs_ref[...], ms)
        bd_ref[...] = jnp.maximum(bd_ref[...], md)


def _tc1(xp, W1, a_src1, a_dst1):
    BLK = 1024
    return pl.pallas_call(
        _tc1_body,
        grid=(NPAD // BLK,),
        in_specs=[
            pl.BlockSpec((BLK, D_IN), lambda i: (i, 0)),
            pl.BlockSpec((D_IN, D_HID), lambda i: (0, 0)),
            pl.BlockSpec((D_HID,), lambda i: (0,)),
            pl.BlockSpec((D_HID,), lambda i: (0,)),
        ],
        out_specs=[
            pl.BlockSpec((BLK, DW), lambda i: (i, 0)),
            pl.BlockSpec((BLK,), lambda i: (i,)),
            pl.BlockSpec((BLK,), lambda i: (i,)),
            pl.BlockSpec((8, 128), lambda i: (0, 0)),
            pl.BlockSpec((8, 128), lambda i: (0, 0)),
        ],
        out_shape=[
            jax.ShapeDtypeStruct((NPAD, DW), _f32),
            jax.ShapeDtypeStruct((NPAD,), _f32),
            jax.ShapeDtypeStruct((NPAD,), _f32),
            jax.ShapeDtypeStruct((8, 128), _f32),
            jax.ShapeDtypeStruct((8, 128), _f32),
        ],
    )(xp, W1, a_src1, a_dst1)


def _sc_edge_body(src_hbm, dst_hbm, htab_hbm, astab_hbm, adtab_hbm,
                  bs_hbm, bd_hbm, zacc_hbm, zden_hbm,
                  acc_out, den_out,
                  src_v, dst_v, as_v, ad_v, pf_v, rows_v, bs_v, bd_v,
                  acc_sh, den_sh):
    cid = lax.axis_index("c")
    sid = lax.axis_index("s")
    wid = cid * NS + sid
    r0 = sid * ROWS_PER_TILE

    pltpu.sync_copy(zacc_hbm.at[pl.ds(r0, ROWS_PER_TILE)],
                    acc_sh.at[pl.ds(r0, ROWS_PER_TILE)])
    pltpu.sync_copy(zden_hbm.at[pl.ds(r0, ROWS_PER_TILE)],
                    den_sh.at[pl.ds(r0, ROWS_PER_TILE)])

    pltpu.sync_copy(src_hbm.at[wid], src_v)
    pltpu.sync_copy(dst_hbm.at[wid], dst_v)
    pltpu.sync_copy(bs_hbm.at[0].at[pl.ds(0, 16)], bs_v)
    pltpu.sync_copy(bd_hbm.at[0].at[pl.ds(0, 16)], bd_v)

    sb = bs_v[...] + bd_v[...]
    bnd = jnp.where(sb >= 0., sb, 0.2 * sb)

    plsc.subcore_barrier()

    def _chunk(c, carry):
        pltpu.sync_copy(astab_hbm.at[src_v.at[c]], as_v.at[c])
        pltpu.sync_copy(adtab_hbm.at[dst_v.at[c]], ad_v.at[c])
        for k in range(CW // 16):
            a = as_v[c, pl.ds(k * 16, 16)]
            d = ad_v[c, pl.ds(k * 16, 16)]
            e = a + d
            e = jnp.where(e >= 0., e, 0.2 * e)
            p = jnp.exp(e - bnd)
            pf_v[pl.ds(k * 16, 16)] = p
        pltpu.sync_copy(pf_v, den_sh.at[dst_v.at[c]], add=True)
        pltpu.sync_copy(htab_hbm.at[src_v.at[c]], rows_v)
        for k in range(CW // 16):
            p16 = pf_v[pl.ds(k * 16, 16)]
            for j in range(16):
                scale = _splat_lane(p16, j)
                r = k * 16 + j
                rows_v[r, :] = rows_v[r, :] * scale
        pltpu.sync_copy(rows_v, acc_sh.at[dst_v.at[c]], add=True)
        return carry

    lax.fori_loop(0, NCH, _chunk, None)

    plsc.subcore_barrier()

    pltpu.sync_copy(acc_sh.at[pl.ds(r0, ROWS_PER_TILE)],
                    acc_out.at[cid].at[pl.ds(r0, ROWS_PER_TILE)])
    pltpu.sync_copy(den_sh.at[pl.ds(r0, ROWS_PER_TILE)],
                    den_out.at[cid].at[pl.ds(r0, ROWS_PER_TILE)])


def _sc_edge(src_r, dst_r, htab, astab, adtab, bs, bd, zacc, zden):
    mesh = plsc.VectorSubcoreMesh(core_axis_name="c", subcore_axis_name="s")
    kfn = pl.kernel(
        _sc_edge_body,
        out_type=[
            jax.ShapeDtypeStruct((2, NPAD, DW), _f32),
            jax.ShapeDtypeStruct((2, NPAD), _f32),
        ],
        mesh=mesh,
        compiler_params=pltpu.CompilerParams(use_tc_tiling_on_sc=False),
        scratch_types=[
            pltpu.VMEM((NCH, CW), _i32),
            pltpu.VMEM((NCH, CW), _i32),
            pltpu.VMEM((NCH, CW), _f32),
            pltpu.VMEM((NCH, CW), _f32),
            pltpu.VMEM((CW,), _f32),
            pltpu.VMEM((CW, DW), _f32),
            pltpu.VMEM((16,), _f32),
            pltpu.VMEM((16,), _f32),
            pltpu.VMEM_SHARED((NPAD, DW), _f32),
            pltpu.VMEM_SHARED((NPAD,), _f32),
        ],
    )
    return kfn(src_r, dst_r, htab, astab, adtab, bs, bd, zacc, zden)


def _tc2_body(acc_ref, den_ref, b1_ref, w2_ref, aws_ref, awd_ref,
              h2_ref, as_ref, ad_ref, bs_ref, bd_ref):
    i = pl.program_id(0)
    a = (acc_ref[0] + acc_ref[1])[:, :D_HID]
    d = den_ref[0] + den_ref[1]
    safe = jnp.where(d > 0., d, 1.)
    y = a / safe[:, None] + b1_ref[...][None, :]
    y = jnp.maximum(y, 0.)
    h2 = jnp.dot(y, w2_ref[...], preferred_element_type=_f32)
    h2_ref[...] = jnp.concatenate(
        [h2, jnp.zeros((h2.shape[0], DW - N_CLASSES), _f32)], axis=1)
    a_s = jnp.sum(h2 * aws_ref[...][None, :], axis=1)
    a_d = jnp.sum(h2 * awd_ref[...][None, :], axis=1)
    as_ref[...] = a_s
    ad_ref[...] = a_d
    ms = jnp.full((8, 128), jnp.max(a_s), _f32)
    md = jnp.full((8, 128), jnp.max(a_d), _f32)

    @pl.when(i == 0)
    def _():
        bs_ref[...] = ms
        bd_ref[...] = md

    @pl.when(i > 0)
    def _():
        bs_ref[...] = jnp.maximum(bs_ref[...], ms)
        bd_ref[...] = jnp.maximum(bd_ref[...], md)


def _tc2(acc1, den1, b1, W2, a_src2, a_dst2):
    BLK = 1024
    return pl.pallas_call(
        _tc2_body,
        grid=(NPAD // BLK,),
        in_specs=[
            pl.BlockSpec((2, BLK, DW), lambda i: (0, i, 0)),
            pl.BlockSpec((2, BLK), lambda i: (0, i)),
            pl.BlockSpec((D_HID,), lambda i: (0,)),
            pl.BlockSpec((D_HID, N_CLASSES), lambda i: (0, 0)),
            pl.BlockSpec((N_CLASSES,), lambda i: (0,)),
            pl.BlockSpec((N_CLASSES,), lambda i: (0,)),
        ],
        out_specs=[
            pl.BlockSpec((BLK, DW), lambda i: (i, 0)),
            pl.BlockSpec((BLK,), lambda i: (i,)),
            pl.BlockSpec((BLK,), lambda i: (i,)),
            pl.BlockSpec((8, 128), lambda i: (0, 0)),
            pl.BlockSpec((8, 128), lambda i: (0, 0)),
        ],
        out_shape=[
            jax.ShapeDtypeStruct((NPAD, DW), _f32),
            jax.ShapeDtypeStruct((NPAD,), _f32),
            jax.ShapeDtypeStruct((NPAD,), _f32),
            jax.ShapeDtypeStruct((8, 128), _f32),
            jax.ShapeDtypeStruct((8, 128), _f32),
        ],
    )(acc1, den1, b1, W2, a_src2, a_dst2)


def _tc3_body(acc_ref, den_ref, b2_ref, out_ref):
    a = acc_ref[0] + acc_ref[1]
    d = den_ref[0] + den_ref[1]
    safe = jnp.where(d > 0., d, 1.)
    o = a / safe[:, None] + b2_ref[...][None, :]
    colmask = lax.broadcasted_iota(_i32, o.shape, 1) < N_CLASSES
    z = jnp.where(colmask, o, -jnp.inf)
    m = jnp.max(z, axis=1, keepdims=True)
    ez = jnp.where(colmask, jnp.exp(z - m), 0.)
    s = jnp.log(jnp.sum(ez, axis=1, keepdims=True))
    out_ref[...] = o - m - s


def _tc3(acc2, den2, b2p):
    BLK = 1024
    return pl.pallas_call(
        _tc3_body,
        grid=(NPAD // BLK,),
        in_specs=[
            pl.BlockSpec((2, BLK, DW), lambda i: (0, i, 0)),
            pl.BlockSpec((2, BLK), lambda i: (0, i)),
            pl.BlockSpec((DW,), lambda i: (0,)),
        ],
        out_specs=pl.BlockSpec((BLK, DW), lambda i: (i, 0)),
        out_shape=jax.ShapeDtypeStruct((NPAD, DW), _f32),
    )(acc2, den2, b2p)


def kernel(x, edge_index, W1, a_src1, a_dst1, b1, W2, a_src2, a_dst2, b2):
    xp = jnp.concatenate(
        [x, jnp.zeros((NPAD - N_NODES, D_IN), _f32)], axis=0)
    pad = PAD_E - N_EDGES
    fill = jnp.full((pad,), DUMMY, _i32)
    src_r = jnp.concatenate([edge_index[0], fill]).reshape(NW, NCH, CW)
    dst_r = jnp.concatenate([edge_index[1], fill]).reshape(NW, NCH, CW)
    z16 = jnp.zeros((NPAD, DW), _f32)
    z1 = jnp.zeros((NPAD,), _f32)

    htab, astab, adtab, bs, bd = _tc1(xp, W1, a_src1, a_dst1)
    acc1, den1 = _sc_edge(src_r, dst_r, htab, astab, adtab, bs, bd, z16, z1)
    h2tab, as2tab, ad2tab, bs2, bd2 = _tc2(acc1, den1, b1, W2,
                                           a_src2, a_dst2)
    acc2, den2 = _sc_edge(src_r, dst_r, h2tab, as2tab, ad2tab, bs2, bd2,
                          z16, z1)
    out = _tc3(acc2, den2, jnp.pad(b2, (0, DW - N_CLASSES)))
    return out[:N_NODES, :N_CLASSES]


# speedup vs baseline: 40.4751x; 1.3756x over previous
import jax
import jax.numpy as jnp
from jax import lax
from jax.experimental import pallas as pl
from jax.experimental.pallas import tpu as pltpu
from jax.experimental.pallas import tpu_sc as plsc

N_NODES = 10000
N_EDGES = 320000
D_IN = 128
D_HID = 8
N_CLASSES = 10

NPAD = 10240
DUMMY = N_NODES
NS = 16
NW = 32
EW = 10240
RCW = 1024
NCH = EW // RCW
PAD_E = NW * EW
ROWS_PER_TILE = NPAD // NS
DW = 16

_f32 = jnp.float32
_i32 = jnp.int32

_GATHER_DNUMS = lax.GatherDimensionNumbers(
    offset_dims=(), collapsed_slice_dims=(0,), start_index_map=(0,))


def _splat_lane(vec16, j):
    idx = jnp.full((16, 1), j, _i32)
    return lax.gather(vec16, idx, _GATHER_DNUMS, slice_sizes=(1,),
                      mode=lax.GatherScatterMode.PROMISE_IN_BOUNDS)


def _tc1_body(x_ref, w_ref, aws_ref, awd_ref,
              h_ref, as_ref, ad_ref, bs_ref, bd_ref):
    i = pl.program_id(0)
    h = jnp.dot(x_ref[...], w_ref[...], preferred_element_type=_f32)
    h_ref[...] = jnp.concatenate(
        [h, jnp.zeros((h.shape[0], DW - D_HID), _f32)], axis=1)
    a_s = jnp.sum(h * aws_ref[...][None, :], axis=1)
    a_d = jnp.sum(h * awd_ref[...][None, :], axis=1)
    as_ref[...] = a_s
    ad_ref[...] = a_d
    ms = jnp.full((8, 128), jnp.max(a_s), _f32)
    md = jnp.full((8, 128), jnp.max(a_d), _f32)

    @pl.when(i == 0)
    def _():
        bs_ref[...] = ms
        bd_ref[...] = md

    @pl.when(i > 0)
    def _():
        bs_ref[...] = jnp.maximum(bs_ref[...], ms)
        bd_ref[...] = jnp.maximum(bd_ref[...], md)


def _tc1(xp, W1, a_src1, a_dst1):
    BLK = 1024
    return pl.pallas_call(
        _tc1_body,
        grid=(NPAD // BLK,),
        in_specs=[
            pl.BlockSpec((BLK, D_IN), lambda i: (i, 0)),
            pl.BlockSpec((D_IN, D_HID), lambda i: (0, 0)),
            pl.BlockSpec((D_HID,), lambda i: (0,)),
            pl.BlockSpec((D_HID,), lambda i: (0,)),
        ],
        out_specs=[
            pl.BlockSpec((BLK, DW), lambda i: (i, 0)),
            pl.BlockSpec((BLK,), lambda i: (i,)),
            pl.BlockSpec((BLK,), lambda i: (i,)),
            pl.BlockSpec((8, 128), lambda i: (0, 0)),
            pl.BlockSpec((8, 128), lambda i: (0, 0)),
        ],
        out_shape=[
            jax.ShapeDtypeStruct((NPAD, DW), _f32),
            jax.ShapeDtypeStruct((NPAD,), _f32),
            jax.ShapeDtypeStruct((NPAD,), _f32),
            jax.ShapeDtypeStruct((8, 128), _f32),
            jax.ShapeDtypeStruct((8, 128), _f32),
        ],
    )(xp, W1, a_src1, a_dst1)


def _sc_edge_body(src_hbm, dst_hbm, htab_hbm, astab_hbm, adtab_hbm,
                  bs_hbm, bd_hbm, zacc_hbm, zden_hbm,
                  acc_out, den_out,
                  src_v, dst_v, a_c, d_c, p_c, rows_v, bs_v, bd_v,
                  acc_sh, den_sh):
    cid = lax.axis_index("c")
    sid = lax.axis_index("s")
    wid = cid * NS + sid
    r0 = sid * ROWS_PER_TILE

    pltpu.sync_copy(zacc_hbm.at[pl.ds(r0, ROWS_PER_TILE)],
                    acc_sh.at[pl.ds(r0, ROWS_PER_TILE)])
    pltpu.sync_copy(zden_hbm.at[pl.ds(r0, ROWS_PER_TILE)],
                    den_sh.at[pl.ds(r0, ROWS_PER_TILE)])

    pltpu.sync_copy(src_hbm.at[wid], src_v)
    pltpu.sync_copy(dst_hbm.at[wid], dst_v)
    pltpu.sync_copy(bs_hbm.at[0].at[pl.ds(0, 16)], bs_v)
    pltpu.sync_copy(bd_hbm.at[0].at[pl.ds(0, 16)], bd_v)

    sb = bs_v[...] + bd_v[...]
    bnd = jnp.where(sb >= 0., sb, 0.2 * sb)

    plsc.subcore_barrier()

    def _chunk(c, carry):
        o = c * RCW
        si = src_v.at[pl.ds(o, RCW)]
        di = dst_v.at[pl.ds(o, RCW)]
        pltpu.sync_copy(astab_hbm.at[si], a_c)
        pltpu.sync_copy(adtab_hbm.at[di], d_c)
        pltpu.sync_copy(htab_hbm.at[si], rows_v)
        for k in range(RCW // 16):
            a = a_c[pl.ds(k * 16, 16)]
            d = d_c[pl.ds(k * 16, 16)]
            e = a + d
            e = jnp.where(e >= 0., e, 0.2 * e)
            p16 = jnp.exp(e - bnd)
            p_c[pl.ds(k * 16, 16)] = p16
            for j in range(16):
                scale = _splat_lane(p16, j)
                r = k * 16 + j
                rows_v[r, :] = rows_v[r, :] * scale
        pltpu.sync_copy(p_c, den_sh.at[di], add=True)
        pltpu.sync_copy(rows_v, acc_sh.at[di], add=True)
        return carry

    lax.fori_loop(0, NCH, _chunk, None)

    plsc.subcore_barrier()

    pltpu.sync_copy(acc_sh.at[pl.ds(r0, ROWS_PER_TILE)],
                    acc_out.at[cid].at[pl.ds(r0, ROWS_PER_TILE)])
    pltpu.sync_copy(den_sh.at[pl.ds(r0, ROWS_PER_TILE)],
                    den_out.at[cid].at[pl.ds(r0, ROWS_PER_TILE)])


def _sc_edge(src_r, dst_r, htab, astab, adtab, bs, bd, zacc, zden):
    mesh = plsc.VectorSubcoreMesh(core_axis_name="c", subcore_axis_name="s")
    kfn = pl.kernel(
        _sc_edge_body,
        out_type=[
            jax.ShapeDtypeStruct((2, NPAD, DW), _f32),
            jax.ShapeDtypeStruct((2, NPAD), _f32),
        ],
        mesh=mesh,
        compiler_params=pltpu.CompilerParams(use_tc_tiling_on_sc=False),
        scratch_types=[
            pltpu.VMEM((EW,), _i32),
            pltpu.VMEM((EW,), _i32),
            pltpu.VMEM((RCW,), _f32),
            pltpu.VMEM((RCW,), _f32),
            pltpu.VMEM((RCW,), _f32),
            pltpu.VMEM((RCW, DW), _f32),
            pltpu.VMEM((16,), _f32),
            pltpu.VMEM((16,), _f32),
            pltpu.VMEM_SHARED((NPAD, DW), _f32),
            pltpu.VMEM_SHARED((NPAD,), _f32),
        ],
    )
    return kfn(src_r, dst_r, htab, astab, adtab, bs, bd, zacc, zden)


def _tc2_body(acc_ref, den_ref, b1_ref, w2_ref, aws_ref, awd_ref,
              h2_ref, as_ref, ad_ref, bs_ref, bd_ref):
    i = pl.program_id(0)
    a = (acc_ref[0] + acc_ref[1])[:, :D_HID]
    d = den_ref[0] + den_ref[1]
    safe = jnp.where(d > 0., d, 1.)
    y = a / safe[:, None] + b1_ref[...][None, :]
    y = jnp.maximum(y, 0.)
    h2 = jnp.dot(y, w2_ref[...], preferred_element_type=_f32)
    h2_ref[...] = jnp.concatenate(
        [h2, jnp.zeros((h2.shape[0], DW - N_CLASSES), _f32)], axis=1)
    a_s = jnp.sum(h2 * aws_ref[...][None, :], axis=1)
    a_d = jnp.sum(h2 * awd_ref[...][None, :], axis=1)
    as_ref[...] = a_s
    ad_ref[...] = a_d
    ms = jnp.full((8, 128), jnp.max(a_s), _f32)
    md = jnp.full((8, 128), jnp.max(a_d), _f32)

    @pl.when(i == 0)
    def _():
        bs_ref[...] = ms
        bd_ref[...] = md

    @pl.when(i > 0)
    def _():
        bs_ref[...] = jnp.maximum(bs_ref[...], ms)
        bd_ref[...] = jnp.maximum(bd_ref[...], md)


def _tc2(acc1, den1, b1, W2, a_src2, a_dst2):
    BLK = 1024
    return pl.pallas_call(
        _tc2_body,
        grid=(NPAD // BLK,),
        in_specs=[
            pl.BlockSpec((2, BLK, DW), lambda i: (0, i, 0)),
            pl.BlockSpec((2, BLK), lambda i: (0, i)),
            pl.BlockSpec((D_HID,), lambda i: (0,)),
            pl.BlockSpec((D_HID, N_CLASSES), lambda i: (0, 0)),
            pl.BlockSpec((N_CLASSES,), lambda i: (0,)),
            pl.BlockSpec((N_CLASSES,), lambda i: (0,)),
        ],
        out_specs=[
            pl.BlockSpec((BLK, DW), lambda i: (i, 0)),
            pl.BlockSpec((BLK,), lambda i: (i,)),
            pl.BlockSpec((BLK,), lambda i: (i,)),
            pl.BlockSpec((8, 128), lambda i: (0, 0)),
            pl.BlockSpec((8, 128), lambda i: (0, 0)),
        ],
        out_shape=[
            jax.ShapeDtypeStruct((NPAD, DW), _f32),
            jax.ShapeDtypeStruct((NPAD,), _f32),
            jax.ShapeDtypeStruct((NPAD,), _f32),
            jax.ShapeDtypeStruct((8, 128), _f32),
            jax.ShapeDtypeStruct((8, 128), _f32),
        ],
    )(acc1, den1, b1, W2, a_src2, a_dst2)


def _tc3_body(acc_ref, den_ref, b2_ref, out_ref):
    a = acc_ref[0] + acc_ref[1]
    d = den_ref[0] + den_ref[1]
    safe = jnp.where(d > 0., d, 1.)
    o = a / safe[:, None] + b2_ref[...][None, :]
    colmask = lax.broadcasted_iota(_i32, o.shape, 1) < N_CLASSES
    z = jnp.where(colmask, o, -jnp.inf)
    m = jnp.max(z, axis=1, keepdims=True)
    ez = jnp.where(colmask, jnp.exp(z - m), 0.)
    s = jnp.log(jnp.sum(ez, axis=1, keepdims=True))
    out_ref[...] = o - m - s


def _tc3(acc2, den2, b2p):
    BLK = 1024
    return pl.pallas_call(
        _tc3_body,
        grid=(NPAD // BLK,),
        in_specs=[
            pl.BlockSpec((2, BLK, DW), lambda i: (0, i, 0)),
            pl.BlockSpec((2, BLK), lambda i: (0, i)),
            pl.BlockSpec((DW,), lambda i: (0,)),
        ],
        out_specs=pl.BlockSpec((BLK, DW), lambda i: (i, 0)),
        out_shape=jax.ShapeDtypeStruct((NPAD, DW), _f32),
    )(acc2, den2, b2p)


def kernel(x, edge_index, W1, a_src1, a_dst1, b1, W2, a_src2, a_dst2, b2):
    xp = jnp.concatenate(
        [x, jnp.zeros((NPAD - N_NODES, D_IN), _f32)], axis=0)
    pad = PAD_E - N_EDGES
    fill = jnp.full((pad,), DUMMY, _i32)
    src_r = jnp.concatenate([edge_index[0], fill]).reshape(NW, EW)
    dst_r = jnp.concatenate([edge_index[1], fill]).reshape(NW, EW)
    z16 = jnp.zeros((NPAD, DW), _f32)
    z1 = jnp.zeros((NPAD,), _f32)

    htab, astab, adtab, bs, bd = _tc1(xp, W1, a_src1, a_dst1)
    acc1, den1 = _sc_edge(src_r, dst_r, htab, astab, adtab, bs, bd, z16, z1)
    h2tab, as2tab, ad2tab, bs2, bd2 = _tc2(acc1, den1, b1, W2,
                                           a_src2, a_dst2)
    acc2, den2 = _sc_edge(src_r, dst_r, h2tab, as2tab, ad2tab, bs2, bd2,
                          z16, z1)
    out = _tc3(acc2, den2, jnp.pad(b2, (0, DW - N_CLASSES)))
    return out[:N_NODES, :N_CLASSES]


# speedup vs baseline: 61.4924x; 1.5193x over previous
import jax
import jax.numpy as jnp
from jax import lax
from jax.experimental import pallas as pl
from jax.experimental.pallas import tpu as pltpu
from jax.experimental.pallas import tpu_sc as plsc

N_NODES = 10000
N_EDGES = 320000
D_IN = 128
D_HID = 8
N_CLASSES = 10

NPAD = 10240
DUMMY = N_NODES
NS = 16
NW = 32
EW = 10240
RCW = 1024
NCH = EW // RCW
PAD_E = NW * EW
ROWS_PER_TILE = NPAD // NS
DW = 16

_f32 = jnp.float32
_i32 = jnp.int32

_GATHER_DNUMS = lax.GatherDimensionNumbers(
    offset_dims=(), collapsed_slice_dims=(0,), start_index_map=(0,))


def _splat_lane(vec16, j):
    idx = jnp.full((16, 1), j, _i32)
    return lax.gather(vec16, idx, _GATHER_DNUMS, slice_sizes=(1,),
                      mode=lax.GatherScatterMode.PROMISE_IN_BOUNDS)


def _tc1_body(x_ref, w_ref, aws_ref, awd_ref,
              h_ref, as_ref, ad_ref, bs_ref, bd_ref):
    i = pl.program_id(0)
    h = jnp.dot(x_ref[...], w_ref[...], preferred_element_type=_f32)
    h_ref[...] = jnp.concatenate(
        [h, jnp.zeros((h.shape[0], DW - D_HID), _f32)], axis=1)
    a_s = jnp.sum(h * aws_ref[...][None, :], axis=1)
    a_d = jnp.sum(h * awd_ref[...][None, :], axis=1)
    as_ref[...] = a_s
    ad_ref[...] = a_d
    ms = jnp.full((8, 128), jnp.max(a_s), _f32)
    md = jnp.full((8, 128), jnp.max(a_d), _f32)

    @pl.when(i == 0)
    def _():
        bs_ref[...] = ms
        bd_ref[...] = md

    @pl.when(i > 0)
    def _():
        bs_ref[...] = jnp.maximum(bs_ref[...], ms)
        bd_ref[...] = jnp.maximum(bd_ref[...], md)


def _tc1(xp, W1, a_src1, a_dst1):
    BLK = 1024
    return pl.pallas_call(
        _tc1_body,
        grid=(NPAD // BLK,),
        in_specs=[
            pl.BlockSpec((BLK, D_IN), lambda i: (i, 0)),
            pl.BlockSpec((D_IN, D_HID), lambda i: (0, 0)),
            pl.BlockSpec((D_HID,), lambda i: (0,)),
            pl.BlockSpec((D_HID,), lambda i: (0,)),
        ],
        out_specs=[
            pl.BlockSpec((BLK, DW), lambda i: (i, 0)),
            pl.BlockSpec((BLK,), lambda i: (i,)),
            pl.BlockSpec((BLK,), lambda i: (i,)),
            pl.BlockSpec((8, 128), lambda i: (0, 0)),
            pl.BlockSpec((8, 128), lambda i: (0, 0)),
        ],
        out_shape=[
            jax.ShapeDtypeStruct((NPAD, DW), _f32),
            jax.ShapeDtypeStruct((NPAD,), _f32),
            jax.ShapeDtypeStruct((NPAD,), _f32),
            jax.ShapeDtypeStruct((8, 128), _f32),
            jax.ShapeDtypeStruct((8, 128), _f32),
        ],
    )(xp, W1, a_src1, a_dst1)


def _sc_edge_body(src_hbm, dst_hbm, htab_hbm, astab_hbm, adtab_hbm,
                  bs_hbm, bd_hbm, zacc_hbm, zden_hbm,
                  acc_out, den_out,
                  src_v, dst_v, a_c, d_c, p_c, rows_v, bs_v, bd_v,
                  acc_sh, den_sh):
    cid = lax.axis_index("c")
    sid = lax.axis_index("s")
    wid = cid * NS + sid
    r0 = sid * ROWS_PER_TILE

    pltpu.sync_copy(zacc_hbm.at[pl.ds(r0, ROWS_PER_TILE)],
                    acc_sh.at[pl.ds(r0, ROWS_PER_TILE)])
    pltpu.sync_copy(zden_hbm.at[pl.ds(r0, ROWS_PER_TILE)],
                    den_sh.at[pl.ds(r0, ROWS_PER_TILE)])

    pltpu.sync_copy(src_hbm.at[wid], src_v)
    pltpu.sync_copy(dst_hbm.at[wid], dst_v)
    pltpu.sync_copy(bs_hbm.at[0].at[pl.ds(0, 16)], bs_v)
    pltpu.sync_copy(bd_hbm.at[0].at[pl.ds(0, 16)], bd_v)

    sb = bs_v[...] + bd_v[...]
    bnd = jnp.where(sb >= 0., sb, 0.2 * sb)

    plsc.subcore_barrier()

    def _chunk(c, carry):
        o = c * RCW
        si = src_v.at[pl.ds(o, RCW)]
        di = dst_v.at[pl.ds(o, RCW)]
        pltpu.sync_copy(astab_hbm.at[si], a_c)
        pltpu.sync_copy(adtab_hbm.at[di], d_c)
        pltpu.sync_copy(htab_hbm.at[si], rows_v)
        for k in range(RCW // 16):
            a = a_c[pl.ds(k * 16, 16)]
            d = d_c[pl.ds(k * 16, 16)]
            e = a + d
            e = jnp.where(e >= 0., e, 0.2 * e)
            p16 = jnp.exp(e - bnd)
            p_c[pl.ds(k * 16, 16)] = p16
            for j in range(16):
                scale = _splat_lane(p16, j)
                r = k * 16 + j
                rows_v[r, :] = rows_v[r, :] * scale
        pltpu.sync_copy(p_c, den_sh.at[di], add=True)
        pltpu.sync_copy(rows_v, acc_sh.at[di], add=True)
        return carry

    lax.fori_loop(0, NCH, _chunk, None)

    plsc.subcore_barrier()

    pltpu.sync_copy(acc_sh.at[pl.ds(r0, ROWS_PER_TILE)],
                    acc_out.at[cid].at[pl.ds(r0, ROWS_PER_TILE)])
    pltpu.sync_copy(den_sh.at[pl.ds(r0, ROWS_PER_TILE)],
                    den_out.at[cid].at[pl.ds(r0, ROWS_PER_TILE)])


def _sc_edge(src_r, dst_r, htab, astab, adtab, bs, bd, zacc, zden):
    mesh = plsc.VectorSubcoreMesh(core_axis_name="c", subcore_axis_name="s")
    kfn = pl.kernel(
        _sc_edge_body,
        out_type=[
            jax.ShapeDtypeStruct((2, NPAD, DW), _f32),
            jax.ShapeDtypeStruct((2, NPAD), _f32),
        ],
        mesh=mesh,
        compiler_params=pltpu.CompilerParams(use_tc_tiling_on_sc=False),
        scratch_types=[
            pltpu.VMEM((EW,), _i32),
            pltpu.VMEM((EW,), _i32),
            pltpu.VMEM((RCW,), _f32),
            pltpu.VMEM((RCW,), _f32),
            pltpu.VMEM((RCW,), _f32),
            pltpu.VMEM((RCW, DW), _f32),
            pltpu.VMEM((16,), _f32),
            pltpu.VMEM((16,), _f32),
            pltpu.VMEM_SHARED((NPAD, DW), _f32),
            pltpu.VMEM_SHARED((NPAD,), _f32),
        ],
    )
    return kfn(src_r, dst_r, htab, astab, adtab, bs, bd, zacc, zden)


def _tc2_body(acc_ref, den_ref, b1_ref, w2_ref, aws_ref, awd_ref,
              h2_ref, as_ref, ad_ref, bs_ref, bd_ref):
    i = pl.program_id(0)
    a = (acc_ref[0] + acc_ref[1])[:, :D_HID]
    d = den_ref[0] + den_ref[1]
    safe = jnp.where(d > 0., d, 1.)
    y = a / safe[:, None] + b1_ref[...][None, :]
    y = jnp.maximum(y, 0.)
    h2 = jnp.dot(y, w2_ref[...], preferred_element_type=_f32)
    h2_ref[...] = jnp.concatenate(
        [h2, jnp.zeros((h2.shape[0], DW - N_CLASSES), _f32)], axis=1)
    a_s = jnp.sum(h2 * aws_ref[...][None, :], axis=1)
    a_d = jnp.sum(h2 * awd_ref[...][None, :], axis=1)
    as_ref[...] = a_s
    ad_ref[...] = a_d
    ms = jnp.full((8, 128), jnp.max(a_s), _f32)
    md = jnp.full((8, 128), jnp.max(a_d), _f32)

    @pl.when(i == 0)
    def _():
        bs_ref[...] = ms
        bd_ref[...] = md

    @pl.when(i > 0)
    def _():
        bs_ref[...] = jnp.maximum(bs_ref[...], ms)
        bd_ref[...] = jnp.maximum(bd_ref[...], md)


def _tc2(acc1, den1, b1, W2, a_src2, a_dst2):
    BLK = 1024
    return pl.pallas_call(
        _tc2_body,
        grid=(NPAD // BLK,),
        in_specs=[
            pl.BlockSpec((2, BLK, DW), lambda i: (0, i, 0)),
            pl.BlockSpec((2, BLK), lambda i: (0, i)),
            pl.BlockSpec((D_HID,), lambda i: (0,)),
            pl.BlockSpec((D_HID, N_CLASSES), lambda i: (0, 0)),
            pl.BlockSpec((N_CLASSES,), lambda i: (0,)),
            pl.BlockSpec((N_CLASSES,), lambda i: (0,)),
        ],
        out_specs=[
            pl.BlockSpec((BLK, DW), lambda i: (i, 0)),
            pl.BlockSpec((BLK,), lambda i: (i,)),
            pl.BlockSpec((BLK,), lambda i: (i,)),
            pl.BlockSpec((8, 128), lambda i: (0, 0)),
            pl.BlockSpec((8, 128), lambda i: (0, 0)),
        ],
        out_shape=[
            jax.ShapeDtypeStruct((NPAD, DW), _f32),
            jax.ShapeDtypeStruct((NPAD,), _f32),
            jax.ShapeDtypeStruct((NPAD,), _f32),
            jax.ShapeDtypeStruct((8, 128), _f32),
            jax.ShapeDtypeStruct((8, 128), _f32),
        ],
    )(acc1, den1, b1, W2, a_src2, a_dst2)


def _tc3_body(acc_ref, den_ref, b2_ref, out_ref):
    a = acc_ref[0] + acc_ref[1]
    d = den_ref[0] + den_ref[1]
    safe = jnp.where(d > 0., d, 1.)
    o = a / safe[:, None] + b2_ref[...][None, :]
    colmask = lax.broadcasted_iota(_i32, o.shape, 1) < N_CLASSES
    z = jnp.where(colmask, o, -jnp.inf)
    m = jnp.max(z, axis=1, keepdims=True)
    ez = jnp.where(colmask, jnp.exp(z - m), 0.)
    s = jnp.log(jnp.sum(ez, axis=1, keepdims=True))
    out_ref[...] = o - m - s


def _tc3(acc2, den2, b2p):
    BLK = 1024
    return pl.pallas_call(
        _tc3_body,
        grid=(NPAD // BLK,),
        in_specs=[
            pl.BlockSpec((2, BLK, DW), lambda i: (0, i, 0)),
            pl.BlockSpec((2, BLK), lambda i: (0, i)),
            pl.BlockSpec((DW,), lambda i: (0,)),
        ],
        out_specs=pl.BlockSpec((BLK, DW), lambda i: (i, 0)),
        out_shape=jax.ShapeDtypeStruct((NPAD, DW), _f32),
    )(acc2, den2, b2p)


def kernel(x, edge_index, W1, a_src1, a_dst1, b1, W2, a_src2, a_dst2, b2):
    xp = jnp.concatenate(
        [x, jnp.zeros((NPAD - N_NODES, D_IN), _f32)], axis=0)
    pad = PAD_E - N_EDGES
    fill = DUMMY + jnp.arange(pad, dtype=_i32) % (NPAD - N_NODES)
    src_r = jnp.concatenate([edge_index[0], fill]).reshape(NW, EW)
    dst_r = jnp.concatenate([edge_index[1], fill]).reshape(NW, EW)
    z16 = jnp.zeros((NPAD, DW), _f32)
    z1 = jnp.zeros((NPAD,), _f32)

    htab, astab, adtab, bs, bd = _tc1(xp, W1, a_src1, a_dst1)
    acc1, den1 = _sc_edge(src_r, dst_r, htab, astab, adtab, bs, bd, z16, z1)
    h2tab, as2tab, ad2tab, bs2, bd2 = _tc2(acc1, den1, b1, W2,
                                           a_src2, a_dst2)
    acc2, den2 = _sc_edge(src_r, dst_r, h2tab, as2tab, ad2tab, bs2, bd2,
                          z16, z1)
    out = _tc3(acc2, den2, jnp.pad(b2, (0, DW - N_CLASSES)))
    return out[:N_NODES, :N_CLASSES]
